# baseline (speedup 1.0000x reference)
import jax
import jax.numpy as jnp
from jax import lax
from jax.experimental import pallas as pl
from jax.experimental.pallas import tpu as pltpu
from jax.experimental.pallas import tpu_sc as plsc

DIM = 64
NUM_WORKERS = 32
CHUNK = 128
BATCH = 4096 * 200
B_PER_W = BATCH // NUM_WORKERS
N_CHUNKS = B_PER_W // CHUNK


def _emb_body(idx_hbm, tab_hbm, out_hbm, idx_v, rows_v, gsem):
    wid = lax.axis_index("s") * 2 + lax.axis_index("c")
    pltpu.sync_copy(idx_hbm.at[wid], idx_v)

    def chunk_step(j, carry):
        pltpu.async_copy(tab_hbm.at[idx_v.at[j]], rows_v, gsem).wait()
        pltpu.sync_copy(rows_v, out_hbm.at[wid, pl.ds(j * CHUNK, CHUNK)])
        return carry

    lax.fori_loop(0, N_CHUNKS, chunk_step, 0)


@jax.jit
def _embed(idx, tab):
    mesh = plsc.VectorSubcoreMesh(core_axis_name="c", subcore_axis_name="s")
    launch = pl.kernel(
        _emb_body,
        out_type=jax.ShapeDtypeStruct((NUM_WORKERS, B_PER_W, DIM), jnp.float32),
        mesh=mesh,
        scratch_types=[
            pltpu.VMEM((N_CHUNKS, CHUNK), jnp.int32),
            pltpu.VMEM((CHUNK, DIM), jnp.float32),
            pltpu.SemaphoreType.DMA,
        ],
        compiler_params=pltpu.CompilerParams(use_tc_tiling_on_sc=False),
    )
    return launch(idx, tab)


def kernel(raw_trg_seq, dst_word_emb):
    idx = raw_trg_seq.reshape(NUM_WORKERS, N_CHUNKS, CHUNK)
    out = _embed(idx, dst_word_emb)
    return out.reshape(4096, 200, DIM)


# speedup vs baseline: 1.1108x; 1.1108x over previous
import jax
import jax.numpy as jnp
from jax import lax
from jax.experimental import pallas as pl
from jax.experimental.pallas import tpu as pltpu
from jax.experimental.pallas import tpu_sc as plsc

DIM = 64
NUM_WORKERS = 32
CHUNK = 128
K = 5
GROUP = K * CHUNK
BATCH = 4096 * 200
B_PER_W = BATCH // NUM_WORKERS
N_CHUNKS = B_PER_W // CHUNK
N_GROUPS = N_CHUNKS // K
N_PAIRS = N_GROUPS // 2


def _emb_body(idx_hbm, tab_hbm, out_hbm, idx_v, rows0, rows1, gsem0, gsem1,
              ssem0, ssem1):
    wid = lax.axis_index("s") * 2 + lax.axis_index("c")
    pltpu.sync_copy(idx_hbm.at[wid], idx_v)

    def fire_gathers(g, buf, sem):
        for j in range(K):
            pltpu.async_copy(tab_hbm.at[idx_v.at[g * K + j]],
                             buf.at[pl.ds(j * CHUNK, CHUNK)], sem)

    def drain(buf, sem):
        pltpu.make_async_copy(tab_hbm.at[pl.ds(0, GROUP)], buf, sem).wait()

    def fire_store(g, buf, sem):
        pltpu.async_copy(buf, out_hbm.at[wid, pl.ds(g * GROUP, GROUP)], sem)

    fire_gathers(0, rows0, gsem0)

    def pair_step(i, carry):
        g0 = 2 * i
        g1 = g0 + 1
        drain(rows0, gsem0)
        fire_store(g0, rows0, ssem0)
        @pl.when(i > 0)
        def _():
            drain(rows1, ssem1)
        fire_gathers(g1, rows1, gsem1)
        drain(rows1, gsem1)
        fire_store(g1, rows1, ssem1)
        drain(rows0, ssem0)
        @pl.when(i + 1 < N_PAIRS)
        def _():
            fire_gathers(g0 + 2, rows0, gsem0)
        return carry

    lax.fori_loop(0, N_PAIRS, pair_step, 0)
    drain(rows1, ssem1)


@jax.jit
def _embed(idx, tab):
    mesh = plsc.VectorSubcoreMesh(core_axis_name="c", subcore_axis_name="s")
    launch = pl.kernel(
        _emb_body,
        out_type=jax.ShapeDtypeStruct((NUM_WORKERS, B_PER_W, DIM), jnp.float32),
        mesh=mesh,
        scratch_types=[
            pltpu.VMEM((N_CHUNKS, CHUNK), jnp.int32),
            pltpu.VMEM((GROUP, DIM), jnp.float32),
            pltpu.VMEM((GROUP, DIM), jnp.float32),
            pltpu.SemaphoreType.DMA,
            pltpu.SemaphoreType.DMA,
            pltpu.SemaphoreType.DMA,
            pltpu.SemaphoreType.DMA,
        ],
        compiler_params=pltpu.CompilerParams(use_tc_tiling_on_sc=False),
    )
    return launch(idx, tab)


def kernel(raw_trg_seq, dst_word_emb):
    idx = raw_trg_seq.reshape(NUM_WORKERS, N_CHUNKS, CHUNK)
    out = _embed(idx, dst_word_emb)
    return out.reshape(4096, 200, DIM)


# speedup vs baseline: 1.1132x; 1.0021x over previous
import jax
import jax.numpy as jnp
from jax import lax
from jax.experimental import pallas as pl
from jax.experimental.pallas import tpu as pltpu
from jax.experimental.pallas import tpu_sc as plsc

DIM = 64
SEQS = 4096
SEQ_LEN = 200
NUM_WORKERS = 32
ROWS_PER_W = SEQS // NUM_WORKERS
CHUNK_SPLITS = ((0, 104), (104, 96))
G_ROWS = 4
N_GROUPS = ROWS_PER_W // G_ROWS
N_PAIRS = N_GROUPS // 2


def _emb_body(idx_hbm, tab_hbm, out_hbm, idx_v, rows0, rows1, gsem0, gsem1,
              ssem0, ssem1):
    wid = lax.axis_index("s") * 2 + lax.axis_index("c")
    row0 = wid * ROWS_PER_W
    pltpu.sync_copy(idx_hbm.at[pl.ds(row0, ROWS_PER_W)], idx_v)

    def fire_gathers(g, buf, sem):
        for ri in range(G_ROWS):
            for off, sz in CHUNK_SPLITS:
                pltpu.async_copy(
                    tab_hbm.at[idx_v.at[g * G_ROWS + ri, pl.ds(off, sz)]],
                    buf.at[ri, pl.ds(off, sz)], sem)

    def drain(sem):
        pltpu.make_async_copy(out_hbm.at[pl.ds(0, G_ROWS)],
                              rows0, sem).wait()

    def fire_store(g, buf, sem):
        pltpu.async_copy(buf, out_hbm.at[pl.ds(row0 + g * G_ROWS, G_ROWS)], sem)

    fire_gathers(0, rows0, gsem0)

    def pair_step(i, carry):
        g0 = 2 * i
        g1 = g0 + 1
        drain(gsem0)
        fire_store(g0, rows0, ssem0)
        @pl.when(i > 0)
        def _():
            drain(ssem1)
        fire_gathers(g1, rows1, gsem1)
        drain(gsem1)
        fire_store(g1, rows1, ssem1)
        drain(ssem0)
        @pl.when(i + 1 < N_PAIRS)
        def _():
            fire_gathers(g0 + 2, rows0, gsem0)
        return carry

    lax.fori_loop(0, N_PAIRS, pair_step, 0)
    drain(ssem1)


@jax.jit
def _embed(idx, tab):
    mesh = plsc.VectorSubcoreMesh(core_axis_name="c", subcore_axis_name="s")
    launch = pl.kernel(
        _emb_body,
        out_type=jax.ShapeDtypeStruct((SEQS, SEQ_LEN, DIM), jnp.float32),
        mesh=mesh,
        scratch_types=[
            pltpu.VMEM((ROWS_PER_W, SEQ_LEN), jnp.int32),
            pltpu.VMEM((G_ROWS, SEQ_LEN, DIM), jnp.float32),
            pltpu.VMEM((G_ROWS, SEQ_LEN, DIM), jnp.float32),
            pltpu.SemaphoreType.DMA,
            pltpu.SemaphoreType.DMA,
            pltpu.SemaphoreType.DMA,
            pltpu.SemaphoreType.DMA,
        ],
        compiler_params=pltpu.CompilerParams(use_tc_tiling_on_sc=False),
    )
    return launch(idx, tab)


def kernel(raw_trg_seq, dst_word_emb):
    return _embed(raw_trg_seq, dst_word_emb)
